# baseline (speedup 1.0000x reference)
import functools

import jax
import jax.numpy as jnp
from jax import lax
from jax.experimental import pallas as pl
from jax.experimental.pallas import tpu as pltpu
from jax.experimental.pallas import tpu_sc as plsc

_TOTAL = 32768
_B = 16
_D = 128
_NC = 2
_NS = 16
_NW = _NC * _NS
_LANES = 16

_R_SC = 4096
_SC_BASE = _TOTAL - _R_SC
_ROWS_PER_W = _R_SC // _NW
_CHUNK = 128
_NCHUNK = _ROWS_PER_W // _CHUNK
_NBUF = 2

_CBLK = 4096
_NCBLK = _TOTAL // _CBLK
_NTCB = _SC_BASE // _CBLK


def _sc_body(x_hbm, seg_hbm, sums_out, xbuf, idxbuf, zbuf, acc_sh,
             lsem, ssem):
    c = lax.axis_index("c")
    s = lax.axis_index("s")
    w = c * _NS + s
    base = _SC_BASE + w * _ROWS_PER_W

    zero_v = jnp.zeros((_LANES,), jnp.float32)

    pltpu.sync_copy(
        seg_hbm.at[pl.ds(_SC_BASE // _CHUNK + w * _NCHUNK, _NCHUNK)], idxbuf)
    off = s * _B
    for g in range(_NCHUNK):
        for k in range(_CHUNK // _LANES):
            sl = pl.ds(k * _LANES, _LANES)
            idxbuf[g, sl] = idxbuf[g, sl] + off

    for i in range(_B):
        for j in range(_D // _LANES):
            zbuf[i, pl.ds(j * _LANES, _LANES)] = zero_v
    pltpu.sync_copy(zbuf, acc_sh.at[pl.ds(off, _B)])

    ld = [None] * _NCHUNK
    sc = [None] * _NCHUNK
    for g in range(min(_NBUF - 1, _NCHUNK)):
        ld[g] = pltpu.async_copy(x_hbm.at[pl.ds(base + g * _CHUNK, _CHUNK)],
                                 xbuf.at[g % _NBUF], lsem.at[g % _NBUF])
    for g in range(_NCHUNK):
        b = g % _NBUF
        ld[g].wait()
        sc[g] = pltpu.async_copy(xbuf.at[b], acc_sh.at[idxbuf.at[g]],
                                 ssem.at[b], add=True)
        nxt = g + _NBUF - 1
        if nxt < _NCHUNK:
            if g >= 1:
                sc[g - 1].wait()
            ld[nxt] = pltpu.async_copy(
                x_hbm.at[pl.ds(base + nxt * _CHUNK, _CHUNK)],
                xbuf.at[nxt % _NBUF], lsem.at[nxt % _NBUF])
    for g in range(max(0, _NCHUNK - _NBUF), _NCHUNK):
        sc[g].wait()

    pltpu.sync_copy(acc_sh.at[pl.ds(off, _B)], sums_out.at[w])


_sc_segment_sums = functools.partial(
    pl.kernel,
    out_type=jax.ShapeDtypeStruct((_NW, _B, _D), jnp.float32),
    mesh=plsc.VectorSubcoreMesh(core_axis_name="c", subcore_axis_name="s",
                                num_cores=_NC, num_subcores=_NS),
    scratch_types=[
        pltpu.VMEM((_NBUF, _CHUNK, _D), jnp.float32),
        pltpu.VMEM((_NCHUNK, _CHUNK), jnp.int32),
        pltpu.VMEM((_B, _D), jnp.float32),
        pltpu.VMEM_SHARED((_NS * _B, _D), jnp.float32),
        pltpu.SemaphoreType.DMA((_NBUF,)),
        pltpu.SemaphoreType.DMA((_NBUF,)),
    ],
)(_sc_body)


def _tc_partial(seg_ref, x_ref, cnt_ref, xsum_ref, acc_c, acc_x):
    i = pl.program_id(0)

    @pl.when(i == 0)
    def _init():
        acc_c[...] = jnp.zeros((_B, _D), jnp.float32)
        acc_x[...] = jnp.zeros((_B, _D), jnp.float32)

    seg = seg_ref[0]
    oh = (lax.broadcasted_iota(jnp.int32, (_B, _CBLK), 0)
          == jnp.broadcast_to(seg, (_B, _CBLK))).astype(jnp.float32)
    acc_c[...] += jnp.broadcast_to(
        jnp.sum(oh, axis=1, keepdims=True), (_B, _D))

    @pl.when(i < _NTCB)
    def _xsum():
        acc_x[...] += lax.dot_general(oh, x_ref[...],
                                      (((1,), (0,)), ((), ())),
                                      precision=lax.Precision.HIGHEST,
                                      preferred_element_type=jnp.float32)

    @pl.when(i == _NCBLK - 1)
    def _out():
        cnt_ref[...] = acc_c[...]
        xsum_ref[...] = acc_x[...]


def _tc_finish(sums_ref, tcsum_ref, cnts_ref, wenc_ref, benc_ref,
               wq0_ref, wq1_ref, keys_ref, q0_ref, q1_ref):
    s = jnp.sum(sums_ref[...], axis=0) + tcsum_ref[...]
    cnt = cnts_ref[...]
    denom = jnp.maximum(cnt, 1.0)
    keys = (jnp.dot(s, wenc_ref[...], preferred_element_type=jnp.float32)
            + cnt * benc_ref[...]) / denom
    keys_ref[...] = keys
    q0_ref[...] = jnp.dot(keys, wq0_ref[...], preferred_element_type=jnp.float32)
    q1_ref[...] = jnp.dot(keys, wq1_ref[...], preferred_element_type=jnp.float32)


def kernel(x, segment_ids, W_enc, b_enc, W_q0, W_q1):
    seg2 = segment_ids.reshape(_TOTAL // _CHUNK, _CHUNK)
    seg3 = segment_ids.reshape(_NCBLK, 1, _CBLK)
    cnts, tcsum = pl.pallas_call(
        _tc_partial,
        grid=(_NCBLK,),
        in_specs=[
            pl.BlockSpec((1, 1, _CBLK), lambda i: (i, 0, 0)),
            pl.BlockSpec((_CBLK, _D),
                         lambda i: (jnp.minimum(i, _NTCB - 1), 0)),
        ],
        out_specs=[pl.BlockSpec((_B, _D), lambda i: (0, 0))] * 2,
        out_shape=[jax.ShapeDtypeStruct((_B, _D), jnp.float32)] * 2,
        scratch_shapes=[pltpu.VMEM((_B, _D), jnp.float32),
                        pltpu.VMEM((_B, _D), jnp.float32)],
        compiler_params=pltpu.CompilerParams(
            dimension_semantics=("arbitrary",)),
    )(seg3, x)
    sums = _sc_segment_sums(x, seg2)
    keys, q0, q1 = pl.pallas_call(
        _tc_finish,
        out_shape=[jax.ShapeDtypeStruct((_B, _D), jnp.float32)] * 3,
    )(sums, tcsum, cnts, W_enc, b_enc.reshape(1, _D), W_q0, W_q1)
    return (keys, q0, q1)


# speedup vs baseline: 1.0901x; 1.0901x over previous
import functools

import jax
import jax.numpy as jnp
from jax import lax
from jax.experimental import pallas as pl
from jax.experimental.pallas import tpu as pltpu
from jax.experimental.pallas import tpu_sc as plsc

_TOTAL = 32768
_B = 16
_D = 128
_NC = 2
_NS = 16
_NW = _NC * _NS
_LANES = 16

_R_SC = 4096
_SC_BASE = _TOTAL - _R_SC
_ROWS_PER_W = _R_SC // _NW
_CHUNK = 128
_NCHUNK = _ROWS_PER_W // _CHUNK
_NBUF = 2

_CBLK = 4096
_NCBLK = _TOTAL // _CBLK
_NTCB = _SC_BASE // _CBLK


def _sc_body(x_hbm, seg_hbm, sums_out, xbuf, idxbuf, zbuf, acc_sh,
             lsem, ssem):
    c = lax.axis_index("c")
    s = lax.axis_index("s")
    w = c * _NS + s
    base = _SC_BASE + w * _ROWS_PER_W

    zero_v = jnp.zeros((_LANES,), jnp.float32)

    pltpu.sync_copy(
        seg_hbm.at[pl.ds(_SC_BASE // _CHUNK + w * _NCHUNK, _NCHUNK)], idxbuf)
    off = s * _B
    for g in range(_NCHUNK):
        for k in range(_CHUNK // _LANES):
            sl = pl.ds(k * _LANES, _LANES)
            idxbuf[g, sl] = idxbuf[g, sl] + off

    for i in range(_B):
        for j in range(_D // _LANES):
            zbuf[i, pl.ds(j * _LANES, _LANES)] = zero_v
    pltpu.sync_copy(zbuf, acc_sh.at[pl.ds(off, _B)])

    ld = [None] * _NCHUNK
    sc = [None] * _NCHUNK
    for g in range(min(_NBUF - 1, _NCHUNK)):
        ld[g] = pltpu.async_copy(x_hbm.at[pl.ds(base + g * _CHUNK, _CHUNK)],
                                 xbuf.at[g % _NBUF], lsem.at[g % _NBUF])
    for g in range(_NCHUNK):
        b = g % _NBUF
        ld[g].wait()
        sc[g] = pltpu.async_copy(xbuf.at[b], acc_sh.at[idxbuf.at[g]],
                                 ssem.at[b], add=True)
        nxt = g + _NBUF - 1
        if nxt < _NCHUNK:
            if g >= 1:
                sc[g - 1].wait()
            ld[nxt] = pltpu.async_copy(
                x_hbm.at[pl.ds(base + nxt * _CHUNK, _CHUNK)],
                xbuf.at[nxt % _NBUF], lsem.at[nxt % _NBUF])
    for g in range(max(0, _NCHUNK - _NBUF), _NCHUNK):
        sc[g].wait()

    plsc.subcore_barrier()
    pltpu.sync_copy(acc_sh.at[pl.ds(off, _B)], zbuf)
    pltpu.sync_copy(zbuf, sums_out.at[w])


_sc_segment_sums = functools.partial(
    pl.kernel,
    out_type=jax.ShapeDtypeStruct((_NW, _B, _D), jnp.float32),
    mesh=plsc.VectorSubcoreMesh(core_axis_name="c", subcore_axis_name="s",
                                num_cores=_NC, num_subcores=_NS),
    scratch_types=[
        pltpu.VMEM((_NBUF, _CHUNK, _D), jnp.float32),
        pltpu.VMEM((_NCHUNK, _CHUNK), jnp.int32),
        pltpu.VMEM((_B, _D), jnp.float32),
        pltpu.VMEM_SHARED((_NS * _B, _D), jnp.float32),
        pltpu.SemaphoreType.DMA((_NBUF,)),
        pltpu.SemaphoreType.DMA((_NBUF,)),
    ],
)(_sc_body)


def _tc_partial(seg_ref, x_ref, cnt_ref, xsum_ref, acc_c, acc_x):
    i = pl.program_id(0)

    @pl.when(i == 0)
    def _init():
        acc_c[...] = jnp.zeros((_B, _D), jnp.float32)
        acc_x[...] = jnp.zeros((_B, _D), jnp.float32)

    seg = seg_ref[0]
    oh = (lax.broadcasted_iota(jnp.int32, (_B, _CBLK), 0)
          == jnp.broadcast_to(seg, (_B, _CBLK))).astype(jnp.float32)
    acc_c[...] += jnp.broadcast_to(
        jnp.sum(oh, axis=1, keepdims=True), (_B, _D))

    @pl.when(i < _NTCB)
    def _xsum():
        acc_x[...] += lax.dot_general(oh, x_ref[...],
                                      (((1,), (0,)), ((), ())),
                                      preferred_element_type=jnp.float32)

    @pl.when(i == _NCBLK - 1)
    def _out():
        cnt_ref[...] = acc_c[...]
        xsum_ref[...] = acc_x[...]


def _tc_finish(sums_ref, tcsum_ref, cnts_ref, wenc_ref, benc_ref,
               wq0_ref, wq1_ref, keys_ref, q0_ref, q1_ref):
    s = jnp.sum(sums_ref[...], axis=0) + tcsum_ref[...]
    cnt = cnts_ref[...]
    denom = jnp.maximum(cnt, 1.0)
    keys = (jnp.dot(s, wenc_ref[...], preferred_element_type=jnp.float32)
            + cnt * benc_ref[...]) / denom
    keys_ref[...] = keys
    q0_ref[...] = jnp.dot(keys, wq0_ref[...], preferred_element_type=jnp.float32)
    q1_ref[...] = jnp.dot(keys, wq1_ref[...], preferred_element_type=jnp.float32)


def kernel(x, segment_ids, W_enc, b_enc, W_q0, W_q1):
    seg2 = segment_ids.reshape(_TOTAL // _CHUNK, _CHUNK)
    seg3 = segment_ids.reshape(_NCBLK, 1, _CBLK)
    cnts, tcsum = pl.pallas_call(
        _tc_partial,
        grid=(_NCBLK,),
        in_specs=[
            pl.BlockSpec((1, 1, _CBLK), lambda i: (i, 0, 0)),
            pl.BlockSpec((_CBLK, _D),
                         lambda i: (jnp.minimum(i, _NTCB - 1), 0)),
        ],
        out_specs=[pl.BlockSpec((_B, _D), lambda i: (0, 0))] * 2,
        out_shape=[jax.ShapeDtypeStruct((_B, _D), jnp.float32)] * 2,
        scratch_shapes=[pltpu.VMEM((_B, _D), jnp.float32),
                        pltpu.VMEM((_B, _D), jnp.float32)],
        compiler_params=pltpu.CompilerParams(
            dimension_semantics=("arbitrary",)),
    )(seg3, x)
    sums = _sc_segment_sums(x, seg2)
    keys, q0, q1 = pl.pallas_call(
        _tc_finish,
        out_shape=[jax.ShapeDtypeStruct((_B, _D), jnp.float32)] * 3,
    )(sums, tcsum, cnts, W_enc, b_enc.reshape(1, _D), W_q0, W_q1)
    return (keys, q0, q1)


# speedup vs baseline: 1.0906x; 1.0005x over previous
import functools

import jax
import jax.numpy as jnp
from jax import lax
from jax.experimental import pallas as pl
from jax.experimental.pallas import tpu as pltpu
from jax.experimental.pallas import tpu_sc as plsc

_TOTAL = 32768
_B = 16
_D = 128
_NC = 2
_NS = 16
_NW = _NC * _NS
_LANES = 16

_R_SC = 4096
_SC_BASE = _TOTAL - _R_SC
_ROWS_PER_W = _R_SC // _NW
_CHUNK = 128
_NCHUNK = _ROWS_PER_W // _CHUNK
_NBUF = 2

_CBLK = 4096
_NCBLK = _TOTAL // _CBLK
_NTCB = _SC_BASE // _CBLK


def _sc_body(x_hbm, seg_hbm, sums_out, xbuf, idxbuf, zbuf, acc_sh,
             lsem, ssem):
    c = lax.axis_index("c")
    s = lax.axis_index("s")
    w = c * _NS + s
    base = _SC_BASE + w * _ROWS_PER_W

    zero_v = jnp.zeros((_LANES,), jnp.float32)

    pltpu.sync_copy(
        seg_hbm.at[pl.ds(_SC_BASE // _CHUNK + w * _NCHUNK, _NCHUNK)], idxbuf)
    off = s * _B
    for g in range(_NCHUNK):
        for k in range(_CHUNK // _LANES):
            sl = pl.ds(k * _LANES, _LANES)
            idxbuf[g, sl] = idxbuf[g, sl] + off

    for i in range(_B):
        for j in range(_D // _LANES):
            zbuf[i, pl.ds(j * _LANES, _LANES)] = zero_v
    pltpu.sync_copy(zbuf, acc_sh.at[pl.ds(off, _B)])

    ld = [None] * _NCHUNK
    sc = [None] * _NCHUNK
    for g in range(min(_NBUF - 1, _NCHUNK)):
        ld[g] = pltpu.async_copy(x_hbm.at[pl.ds(base + g * _CHUNK, _CHUNK)],
                                 xbuf.at[g % _NBUF], lsem.at[g % _NBUF])
    for g in range(_NCHUNK):
        b = g % _NBUF
        ld[g].wait()
        sc[g] = pltpu.async_copy(xbuf.at[b], acc_sh.at[idxbuf.at[g]],
                                 ssem.at[b], add=True)
        nxt = g + _NBUF - 1
        if nxt < _NCHUNK:
            if g >= 1:
                sc[g - 1].wait()
            ld[nxt] = pltpu.async_copy(
                x_hbm.at[pl.ds(base + nxt * _CHUNK, _CHUNK)],
                xbuf.at[nxt % _NBUF], lsem.at[nxt % _NBUF])
    for g in range(max(0, _NCHUNK - _NBUF), _NCHUNK):
        sc[g].wait()

    plsc.subcore_barrier()
    plsc.subcore_barrier()
    pltpu.sync_copy(acc_sh.at[pl.ds(off, _B)], zbuf)
    pltpu.sync_copy(zbuf, sums_out.at[w])


_sc_segment_sums = functools.partial(
    pl.kernel,
    out_type=jax.ShapeDtypeStruct((_NW, _B, _D), jnp.float32),
    mesh=plsc.VectorSubcoreMesh(core_axis_name="c", subcore_axis_name="s",
                                num_cores=_NC, num_subcores=_NS),
    scratch_types=[
        pltpu.VMEM((_NBUF, _CHUNK, _D), jnp.float32),
        pltpu.VMEM((_NCHUNK, _CHUNK), jnp.int32),
        pltpu.VMEM((_B, _D), jnp.float32),
        pltpu.VMEM_SHARED((_NS * _B, _D), jnp.float32),
        pltpu.SemaphoreType.DMA((_NBUF,)),
        pltpu.SemaphoreType.DMA((_NBUF,)),
    ],
)(_sc_body)


def _tc_partial(seg_ref, x_ref, cnt_ref, xsum_ref, acc_c, acc_x):
    i = pl.program_id(0)

    @pl.when(i == 0)
    def _init():
        acc_c[...] = jnp.zeros((_B, _D), jnp.float32)
        acc_x[...] = jnp.zeros((_B, _D), jnp.float32)

    seg = seg_ref[0]
    oh = (lax.broadcasted_iota(jnp.int32, (_B, _CBLK), 0)
          == jnp.broadcast_to(seg, (_B, _CBLK))).astype(jnp.float32)
    acc_c[...] += jnp.broadcast_to(
        jnp.sum(oh, axis=1, keepdims=True), (_B, _D))

    @pl.when(i < _NTCB)
    def _xsum():
        acc_x[...] += lax.dot_general(oh, x_ref[...],
                                      (((1,), (0,)), ((), ())),
                                      preferred_element_type=jnp.float32)

    @pl.when(i == _NCBLK - 1)
    def _out():
        cnt_ref[...] = acc_c[...]
        xsum_ref[...] = acc_x[...]


def _tc_finish(sums_ref, tcsum_ref, cnts_ref, wenc_ref, benc_ref,
               wq0_ref, wq1_ref, keys_ref, q0_ref, q1_ref):
    s = jnp.sum(sums_ref[...], axis=0) + tcsum_ref[...]
    cnt = cnts_ref[...]
    denom = jnp.maximum(cnt, 1.0)
    keys = (jnp.dot(s, wenc_ref[...], preferred_element_type=jnp.float32)
            + cnt * benc_ref[...]) / denom
    keys_ref[...] = keys
    q0_ref[...] = jnp.dot(keys, wq0_ref[...], preferred_element_type=jnp.float32)
    q1_ref[...] = jnp.dot(keys, wq1_ref[...], preferred_element_type=jnp.float32)


def kernel(x, segment_ids, W_enc, b_enc, W_q0, W_q1):
    seg2 = segment_ids.reshape(_TOTAL // _CHUNK, _CHUNK)
    seg3 = segment_ids.reshape(_NCBLK, 1, _CBLK)
    cnts, tcsum = pl.pallas_call(
        _tc_partial,
        grid=(_NCBLK,),
        in_specs=[
            pl.BlockSpec((1, 1, _CBLK), lambda i: (i, 0, 0)),
            pl.BlockSpec((_CBLK, _D),
                         lambda i: (jnp.minimum(i, _NTCB - 1), 0)),
        ],
        out_specs=[pl.BlockSpec((_B, _D), lambda i: (0, 0))] * 2,
        out_shape=[jax.ShapeDtypeStruct((_B, _D), jnp.float32)] * 2,
        scratch_shapes=[pltpu.VMEM((_B, _D), jnp.float32),
                        pltpu.VMEM((_B, _D), jnp.float32)],
        compiler_params=pltpu.CompilerParams(
            dimension_semantics=("arbitrary",)),
    )(seg3, x)
    sums = _sc_segment_sums(x, seg2)
    keys, q0, q1 = pl.pallas_call(
        _tc_finish,
        out_shape=[jax.ShapeDtypeStruct((_B, _D), jnp.float32)] * 3,
    )(sums, tcsum, cnts, W_enc, b_enc.reshape(1, _D), W_q0, W_q1)
    return (keys, q0, q1)
